# baseline (speedup 1.0000x reference)
import functools
import math

import jax
import jax.numpy as jnp
from jax import lax
from jax.experimental import pallas as pl
from jax.experimental.pallas import tpu as pltpu
from jax.experimental.pallas import tpu_sc as plsc

_VOCAB = 1000000
_EMBED = 64
_BATCH = 4096
_HIST = 200
_LANES = 16

_B_TOTAL = _BATCH * _HIST
_BLK = 128
_N_BLOCKS = _B_TOTAL // _BLK
_NW = 32
_BLOCKS_PER_W = _N_BLOCKS // _NW
_SUB = 4
_STEPS = _BLOCKS_PER_W // _SUB
_SCALE = math.sqrt(_EMBED)


def _emb_body(x_hbm, table_hbm, out_hbm, idx_v, rows_v, sem):
    wid = lax.axis_index("s") * 2 + lax.axis_index("c")
    base = wid * _BLOCKS_PER_W

    def step(k, carry):
        off = base + k * _SUB
        pltpu.sync_copy(x_hbm.at[pl.ds(off, _SUB)], idx_v)
        copies = [
            pltpu.async_copy(table_hbm.at[idx_v.at[j]], rows_v.at[j], sem)
            for j in range(_SUB)
        ]
        for cp in copies:
            cp.wait()

        def scale_row(r, c2):
            for j in range(_SUB):
                for c in range(_EMBED // _LANES):
                    sl = pl.ds(c * _LANES, _LANES)
                    rows_v[j, r, sl] = rows_v[j, r, sl] * _SCALE
            return c2

        lax.fori_loop(0, _BLK, scale_row, 0)
        pltpu.sync_copy(rows_v, out_hbm.at[pl.ds(off, _SUB)])
        return carry

    lax.fori_loop(0, _STEPS, step, 0)


def kernel(x, table):
    xf = x.reshape(_N_BLOCKS, _BLK)
    run = pl.kernel(
        _emb_body,
        out_type=jax.ShapeDtypeStruct((_N_BLOCKS, _BLK, _EMBED), jnp.float32),
        mesh=plsc.VectorSubcoreMesh(core_axis_name="c", subcore_axis_name="s"),
        scratch_types=[
            pltpu.VMEM((_SUB, _BLK), jnp.int32),
            pltpu.VMEM((_SUB, _BLK, _EMBED), jnp.float32),
            pltpu.SemaphoreType.DMA,
        ],
        compiler_params=pltpu.CompilerParams(use_tc_tiling_on_sc=False),
    )
    out = run(xf, table)
    return out.reshape(_BATCH, _HIST, _EMBED)


# speedup vs baseline: 1.0840x; 1.0840x over previous
import math

import jax
import jax.numpy as jnp
from jax import lax
from jax.experimental import pallas as pl
from jax.experimental.pallas import tpu as pltpu
from jax.experimental.pallas import tpu_sc as plsc

_VOCAB = 1000000
_EMBED = 64
_BATCH = 4096
_HIST = 200
_LANES = 16

_B_TOTAL = _BATCH * _HIST
_BLK = 128
_N_BLOCKS = _B_TOTAL // _BLK
_NW = 32
_BLOCKS_PER_W = _N_BLOCKS // _NW
_NBUF = 4
_AHEAD = 2
_SCALE = math.sqrt(_EMBED)


def _emb_body(x_hbm, table_hbm, out_hbm, idx_all, rows_v, gsem, wsem):
    wid = lax.axis_index("s") * 2 + lax.axis_index("c")
    base = wid * _BLOCKS_PER_W
    pltpu.sync_copy(x_hbm.at[pl.ds(base, _BLOCKS_PER_W)], idx_all)

    def fire_gather(s, b):
        pltpu.async_copy(table_hbm.at[idx_all.at[s]], rows_v.at[b], gsem.at[b])

    for s0 in range(_AHEAD):
        fire_gather(s0, s0)

    def outer(g, carry):
        for b in range(_NBUF):
            s = g * _NBUF + b
            bn = (b + _AHEAD) % _NBUF
            pltpu.make_async_copy(
                table_hbm.at[idx_all.at[s]], rows_v.at[b], gsem.at[b]
            ).wait()

            @pl.when(jnp.logical_and(s >= _NBUF - _AHEAD, s < _BLOCKS_PER_W - _AHEAD))
            def _drain():
                pltpu.make_async_copy(
                    rows_v.at[bn], out_hbm.at[base], wsem.at[bn]
                ).wait()

            @pl.when(s < _BLOCKS_PER_W - _AHEAD)
            def _prefetch():
                fire_gather(s + _AHEAD, bn)

            @plsc.parallel_loop(0, _BLK, step=1, unroll=4)
            def _scale(r):
                for c in range(_EMBED // _LANES):
                    sl = pl.ds(c * _LANES, _LANES)
                    rows_v[b, r, sl] = rows_v[b, r, sl] * _SCALE

            pltpu.async_copy(rows_v.at[b], out_hbm.at[base + s], wsem.at[b])
        return carry

    lax.fori_loop(0, _BLOCKS_PER_W // _NBUF, outer, 0)

    for b in range(_NBUF):
        pltpu.make_async_copy(rows_v.at[b], out_hbm.at[base], wsem.at[b]).wait()


def kernel(x, table):
    xf = x.reshape(_N_BLOCKS, _BLK)
    run = pl.kernel(
        _emb_body,
        out_type=jax.ShapeDtypeStruct((_N_BLOCKS, _BLK, _EMBED), jnp.float32),
        mesh=plsc.VectorSubcoreMesh(core_axis_name="c", subcore_axis_name="s"),
        scratch_types=[
            pltpu.VMEM((_BLOCKS_PER_W, _BLK), jnp.int32),
            pltpu.VMEM((_NBUF, _BLK, _EMBED), jnp.float32),
            pltpu.SemaphoreType.DMA((_NBUF,)),
            pltpu.SemaphoreType.DMA((_NBUF,)),
        ],
        compiler_params=pltpu.CompilerParams(use_tc_tiling_on_sc=False),
    )
    out = run(xf, table)
    return out.reshape(_BATCH, _HIST, _EMBED)


# speedup vs baseline: 1.0868x; 1.0027x over previous
import math

import jax
import jax.numpy as jnp
from jax import lax
from jax.experimental import pallas as pl
from jax.experimental.pallas import tpu as pltpu
from jax.experimental.pallas import tpu_sc as plsc

_VOCAB = 1000000
_EMBED = 64
_BATCH = 4096
_HIST = 200
_LANES = 16

_B_TOTAL = _BATCH * _HIST
_BLK = 128
_N_BLOCKS = _B_TOTAL // _BLK
_NW = 32
_BLOCKS_PER_W = _N_BLOCKS // _NW
_NBUF = 4
_AHEAD = 2
_SCALE = math.sqrt(_EMBED)


def _emb_body(x_hbm, table_hbm, out_hbm, idx_all, rows_v, gsem, wsem):
    wid = lax.axis_index("s") * 2 + lax.axis_index("c")
    base = wid * _BLOCKS_PER_W
    pltpu.sync_copy(x_hbm.at[pl.ds(base, _BLOCKS_PER_W)], idx_all)

    def fire_gather(s, b):
        pltpu.async_copy(table_hbm.at[idx_all.at[s]], rows_v.at[b], gsem.at[b])

    def out_rows(s):
        return out_hbm.at[pl.ds((base + s) * _BLK, _BLK)]

    for s0 in range(_AHEAD):
        fire_gather(s0, s0)

    def outer(g, carry):
        for b in range(_NBUF):
            s = g * _NBUF + b
            bn = (b + _AHEAD) % _NBUF
            pltpu.make_async_copy(
                table_hbm.at[idx_all.at[s]], rows_v.at[b], gsem.at[b]
            ).wait()

            @pl.when(jnp.logical_and(s >= _NBUF - _AHEAD, s < _BLOCKS_PER_W - _AHEAD))
            def _drain():
                pltpu.make_async_copy(
                    rows_v.at[bn], out_rows(0), wsem.at[bn]
                ).wait()

            @pl.when(s < _BLOCKS_PER_W - _AHEAD)
            def _prefetch():
                fire_gather(s + _AHEAD, bn)

            @plsc.parallel_loop(0, _BLK, step=1, unroll=4)
            def _scale(r):
                for c in range(_EMBED // _LANES):
                    sl = pl.ds(c * _LANES, _LANES)
                    rows_v[b, r, sl] = rows_v[b, r, sl] * _SCALE

            pltpu.async_copy(rows_v.at[b], out_rows(s), wsem.at[b])
        return carry

    lax.fori_loop(0, _BLOCKS_PER_W // _NBUF, outer, 0)

    for b in range(_NBUF):
        pltpu.make_async_copy(rows_v.at[b], out_rows(0), wsem.at[b]).wait()


def kernel(x, table):
    xf = x.reshape(_N_BLOCKS, _BLK)
    run = pl.kernel(
        _emb_body,
        out_type=jax.ShapeDtypeStruct((_B_TOTAL, _EMBED), jnp.float32),
        mesh=plsc.VectorSubcoreMesh(core_axis_name="c", subcore_axis_name="s"),
        scratch_types=[
            pltpu.VMEM((_BLOCKS_PER_W, _BLK), jnp.int32),
            pltpu.VMEM((_NBUF, _BLK, _EMBED), jnp.float32),
            pltpu.SemaphoreType.DMA((_NBUF,)),
            pltpu.SemaphoreType.DMA((_NBUF,)),
        ],
        compiler_params=pltpu.CompilerParams(use_tc_tiling_on_sc=False),
    )
    out = run(xf, table)
    return out.reshape(_BATCH, _HIST, _EMBED)


# speedup vs baseline: 1.2628x; 1.1619x over previous
import math

import jax
import jax.numpy as jnp
from jax import lax
from jax.experimental import layout as jax_layout
from jax.experimental import pallas as pl
from jax.experimental.pallas import tpu as pltpu
from jax.experimental.pallas import tpu_sc as plsc

_VOCAB = 1000000
_EMBED = 64
_BATCH = 4096
_HIST = 200
_LANES = 16

_B_TOTAL = _BATCH * _HIST
_BLK = 128
_N_BLOCKS = _B_TOTAL // _BLK
_NW = 32
_BLOCKS_PER_W = _N_BLOCKS // _NW
_NBUF = 4
_AHEAD = 2
_SCALE = math.sqrt(_EMBED)


def _emb_body(x_hbm, table_hbm, out_hbm, idx_all, rows_v, gsem, wsem):
    wid = lax.axis_index("s") * 2 + lax.axis_index("c")
    base = wid * _BLOCKS_PER_W
    pltpu.sync_copy(x_hbm.at[pl.ds(base, _BLOCKS_PER_W)], idx_all)

    def fire_gather(s, b):
        pltpu.async_copy(table_hbm.at[idx_all.at[s]], rows_v.at[b], gsem.at[b])

    def out_rows(s):
        return out_hbm.at[pl.ds((base + s) * _BLK, _BLK)]

    for s0 in range(_AHEAD):
        fire_gather(s0, s0)

    def outer(g, carry):
        for b in range(_NBUF):
            s = g * _NBUF + b
            bn = (b + _AHEAD) % _NBUF
            pltpu.make_async_copy(
                table_hbm.at[idx_all.at[s]], rows_v.at[b], gsem.at[b]
            ).wait()

            @pl.when(jnp.logical_and(s >= _NBUF - _AHEAD, s < _BLOCKS_PER_W - _AHEAD))
            def _drain():
                pltpu.make_async_copy(
                    rows_v.at[bn], out_rows(0), wsem.at[bn]
                ).wait()

            @pl.when(s < _BLOCKS_PER_W - _AHEAD)
            def _prefetch():
                fire_gather(s + _AHEAD, bn)

            @plsc.parallel_loop(0, _BLK, step=1, unroll=4)
            def _scale(r):
                for c in range(_EMBED // _LANES):
                    sl = pl.ds(c * _LANES, _LANES)
                    rows_v[b, r, sl] = rows_v[b, r, sl] * _SCALE

            pltpu.async_copy(rows_v.at[b], out_rows(s), wsem.at[b])
        return carry

    lax.fori_loop(0, _BLOCKS_PER_W // _NBUF, outer, 0)

    for b in range(_NBUF):
        pltpu.make_async_copy(rows_v.at[b], out_rows(0), wsem.at[b]).wait()


def kernel(x, table):
    xf = x.reshape(_N_BLOCKS, _BLK)
    run = pl.kernel(
        _emb_body,
        out_type=jax.ShapeDtypeStruct((_B_TOTAL, _EMBED), jnp.float32),
        mesh=plsc.VectorSubcoreMesh(core_axis_name="c", subcore_axis_name="s"),
        scratch_types=[
            pltpu.VMEM((_BLOCKS_PER_W, _BLK), jnp.int32),
            pltpu.VMEM((_NBUF, _BLK, _EMBED), jnp.float32),
            pltpu.SemaphoreType.DMA((_NBUF,)),
            pltpu.SemaphoreType.DMA((_NBUF,)),
        ],
        compiler_params=pltpu.CompilerParams(use_tc_tiling_on_sc=False),
    )
    out = run(xf, table)
    out = out.reshape(_BATCH, _HIST, _EMBED)
    return jax_layout.with_layout_constraint(
        out, jax_layout.Layout((0, 1, 2), tiling=((1024,),))
    )
